# Initial kernel scaffold; baseline (speedup 1.0000x reference)
import jax
import jax.numpy as jnp
from jax.experimental import pallas as pl


def kernel(x, edge_index, batch, Wroot, Wnbr, bconv, gamma, beta, Ws, bs, hW1, hb1, hW2, hb2, hW3, hb3):
    raise NotImplementedError("write your pallas kernel here")


# speedup vs baseline: 4.3371x; 4.3371x over previous
import functools

import jax
import jax.numpy as jnp
from jax import lax
from jax.experimental import pallas as pl
from jax.experimental.pallas import tpu as pltpu
from jax.experimental.pallas import tpu_sc as plsc

N = 10000
E = 160000
D = 256
DH = D // 2
G = 64
LAYERS = 3

NC = 2
NS = 16
LANES = 16

N_PAD = 10240
ROWS_PT = N_PAD // NS
EPT = E // NS
CH = 80
NCHUNK = EPT // CH
NCG = 25
NGRP = NCHUNK // NCG
ZR = 64

BLK = 400
NB = N // BLK

_sc_mesh = plsc.VectorSubcoreMesh(
    core_axis_name="c", subcore_axis_name="s", num_cores=NC, num_subcores=NS)


@functools.partial(
    pl.kernel,
    out_type=jax.ShapeDtypeStruct((N_PAD,), jnp.float32),
    mesh=_sc_mesh,
    compiler_params=pltpu.CompilerParams(needs_layout_passes=False),
    scratch_types=[
        pltpu.VMEM((NCHUNK, CH), jnp.int32),
        pltpu.VMEM((N_PAD,), jnp.float32),
        pltpu.VMEM((NS, ROWS_PT), jnp.float32),
        pltpu.VMEM_SHARED((NS, NS, ROWS_PT), jnp.float32),
    ],
)
def _deg_kernel(dst_hbm, out_hbm, dst_v, deg_v, red_v, stage_sh):
    c = lax.axis_index("c")
    s = lax.axis_index("s")
    zf = jnp.zeros((LANES,), jnp.float32)
    ones = jnp.ones((LANES,), jnp.float32)

    def zbody(i, carry):
        deg_v[pl.ds(i * LANES, LANES)] = zf
        return carry
    lax.fori_loop(0, N_PAD // LANES, zbody, 0)

    pltpu.sync_copy(dst_hbm.at[s], dst_v)

    def ebody(j, carry):
        def inner(k, carry2):
            idx = dst_v[j, pl.ds(k * LANES, LANES)]
            plsc.addupdate_scatter(deg_v, [idx], ones)
            return carry2
        return lax.fori_loop(0, CH // LANES, inner, carry)
    lax.fori_loop(0, NCHUNK, ebody, 0)

    for k in range(NS):
        pltpu.sync_copy(deg_v.at[pl.ds(k * ROWS_PT, ROWS_PT)], stage_sh.at[k, s])
    plsc.subcore_barrier()

    pltpu.sync_copy(stage_sh.at[s], red_v)

    def rbody(j, carry):
        acc = zf
        for r in range(NS):
            acc = acc + red_v[r, pl.ds(j * LANES, LANES)]
        acc = jnp.maximum(acc, 1.0)
        deg_v[pl.ds(j * LANES, LANES)] = 1.0 / acc
        return carry
    lax.fori_loop(0, ROWS_PT // LANES, rbody, 0)

    @pl.when(c == 0)
    def _():
        pltpu.sync_copy(deg_v.at[pl.ds(0, ROWS_PT)],
                        out_hbm.at[pl.ds(s * ROWS_PT, ROWS_PT)])


@functools.partial(
    pl.kernel,
    out_type=(jax.ShapeDtypeStruct((N_PAD, DH), jnp.float32),
              jax.ShapeDtypeStruct((N_PAD, DH), jnp.float32)),
    mesh=_sc_mesh,
    compiler_params=pltpu.CompilerParams(needs_layout_passes=False),
    scratch_types=[
        pltpu.VMEM((NCG, CH), jnp.int32),
        pltpu.VMEM((NCG, CH), jnp.int32),
        pltpu.VMEM((CH, DH), jnp.float32),
        pltpu.VMEM((ZR, DH), jnp.float32),
        pltpu.VMEM_SHARED((N_PAD, DH), jnp.float32),
        pltpu.SemaphoreType.DMA,
    ],
)
def _msg_kernel(h0_hbm, h1_hbm, src_hbm, dst_hbm, out0_hbm, out1_hbm,
                src_v, dst_v, rows_v, zero_v, acc_sh, sem):
    c = lax.axis_index("c")
    s = lax.axis_index("s")
    zf = jnp.zeros((LANES,), jnp.float32)

    def zrow(i, carry):
        def zcol(j, carry2):
            zero_v[i, pl.ds(j * LANES, LANES)] = zf
            return carry2
        return lax.fori_loop(0, DH // LANES, zcol, carry)
    lax.fori_loop(0, ZR, zrow, 0)

    base = s * ROWS_PT
    for k in range(ROWS_PT // ZR):
        pltpu.sync_copy(zero_v, acc_sh.at[pl.ds(base + k * ZR, ZR)])

    plsc.subcore_barrier()

    def make_loop(h_hbm):
        def run():
            for g in range(NGRP):
                pltpu.sync_copy(src_hbm.at[s, g], src_v)
                pltpu.sync_copy(dst_hbm.at[s, g], dst_v)

                def chunk(j, carry):
                    pltpu.async_copy(h_hbm.at[src_v.at[j]], rows_v, sem).wait()
                    pltpu.sync_copy(rows_v, acc_sh.at[dst_v.at[j]], add=True)
                    return carry
                lax.fori_loop(0, NCG, chunk, 0)
        return run

    @pl.when(c == 0)
    def _():
        make_loop(h0_hbm)()

    @pl.when(c == 1)
    def _():
        make_loop(h1_hbm)()

    plsc.subcore_barrier()

    @pl.when(c == 0)
    def _():
        pltpu.sync_copy(acc_sh.at[pl.ds(base, ROWS_PT)],
                        out0_hbm.at[pl.ds(base, ROWS_PT)])

    @pl.when(c == 1)
    def _():
        pltpu.sync_copy(acc_sh.at[pl.ds(base, ROWS_PT)],
                        out1_hbm.at[pl.ds(base, ROWS_PT)])


def _layer_a_body(h0, h1, m0, m1, dinv, wcat, bc, z, ssum, ssq):
    i = pl.program_id(0)
    di = dinv[...]
    hm = jnp.concatenate(
        [h0[...], h1[...], m0[...] * di, m1[...] * di], axis=1)
    zz = jnp.dot(hm, wcat[...], preferred_element_type=jnp.float32) + bc[...]
    z[...] = zz

    @pl.when(i == 0)
    def _():
        ssum[...] = jnp.zeros_like(ssum)
        ssq[...] = jnp.zeros_like(ssq)

    ssum[...] += jnp.sum(zz, axis=0, keepdims=True)
    ssq[...] += jnp.sum(zz * zz, axis=0, keepdims=True)


def _layer_a(h0, h1, m0, m1, dinv, wcat, bc):
    return pl.pallas_call(
        _layer_a_body,
        grid=(NB,),
        in_specs=[
            pl.BlockSpec((BLK, DH), lambda i: (i, 0)),
            pl.BlockSpec((BLK, DH), lambda i: (i, 0)),
            pl.BlockSpec((BLK, DH), lambda i: (i, 0)),
            pl.BlockSpec((BLK, DH), lambda i: (i, 0)),
            pl.BlockSpec((BLK, 1), lambda i: (i, 0)),
            pl.BlockSpec((2 * D, D), lambda i: (0, 0)),
            pl.BlockSpec((1, D), lambda i: (0, 0)),
        ],
        out_specs=[
            pl.BlockSpec((BLK, D), lambda i: (i, 0)),
            pl.BlockSpec((1, D), lambda i: (0, 0)),
            pl.BlockSpec((1, D), lambda i: (0, 0)),
        ],
        out_shape=[
            jax.ShapeDtypeStruct((N, D), jnp.float32),
            jax.ShapeDtypeStruct((1, D), jnp.float32),
            jax.ShapeDtypeStruct((1, D), jnp.float32),
        ],
    )(h0, h1, m0, m1, dinv, wcat, bc)


def _layer_b_body(z, ssum, ssq, gam, bet, h0, h1):
    mu = ssum[...] * (1.0 / N)
    var = ssq[...] * (1.0 / N) - mu * mu
    y = (z[...] - mu) / jnp.sqrt(var + 1e-5) * gam[...] + bet[...]
    y = jnp.maximum(y, 0.0)
    h0[...] = y[:, :DH]
    h1[...] = y[:, DH:]


def _layer_b(z, ssum, ssq, gam, bet):
    return pl.pallas_call(
        _layer_b_body,
        grid=(NB,),
        in_specs=[
            pl.BlockSpec((BLK, D), lambda i: (i, 0)),
            pl.BlockSpec((1, D), lambda i: (0, 0)),
            pl.BlockSpec((1, D), lambda i: (0, 0)),
            pl.BlockSpec((1, D), lambda i: (0, 0)),
            pl.BlockSpec((1, D), lambda i: (0, 0)),
        ],
        out_specs=[
            pl.BlockSpec((BLK, DH), lambda i: (i, 0)),
            pl.BlockSpec((BLK, DH), lambda i: (i, 0)),
        ],
        out_shape=[
            jax.ShapeDtypeStruct((N, DH), jnp.float32),
            jax.ShapeDtypeStruct((N, DH), jnp.float32),
        ],
    )(z, ssum, ssq, gam, bet)


def _pool_body(h0, h1, bat, ws0, ws1, bs0, bs1,
               w10, b10, w20, b20, w30, b30,
               w11, b11, w21, b21, w31, b31,
               w12, b12, w22, b22, w32, b32,
               out, gsum, gcnt):
    i = pl.program_id(0)

    @pl.when(i == 0)
    def _():
        gsum[...] = jnp.zeros_like(gsum)
        gcnt[...] = jnp.zeros_like(gcnt)

    h = jnp.concatenate([h0[...], h1[...]], axis=1)
    onehot = (bat[...] == lax.broadcasted_iota(jnp.int32, (1, G), 1)
              ).astype(jnp.float32)
    gsum[...] += lax.dot_general(onehot, h, (((0,), (0,)), ((), ())),
                                 preferred_element_type=jnp.float32)
    gcnt[...] += lax.dot_general(onehot, jnp.ones((BLK, 1), jnp.float32),
                                 (((0,), (0,)), ((), ())),
                                 preferred_element_type=jnp.float32)

    @pl.when(i == NB - 1)
    def _():
        g = gsum[...] / jnp.maximum(gcnt[...], 1.0)
        g = jnp.maximum(jnp.dot(g, ws0[...],
                                preferred_element_type=jnp.float32) + bs0[...], 0.0)
        g = jnp.maximum(jnp.dot(g, ws1[...],
                                preferred_element_type=jnp.float32) + bs1[...], 0.0)
        outs = []
        for w1, b1, w2, b2, w3, b3 in ((w10, b10, w20, b20, w30, b30),
                                       (w11, b11, w21, b21, w31, b31),
                                       (w12, b12, w22, b22, w32, b32)):
            t = jnp.maximum(jnp.dot(g, w1[...],
                                    preferred_element_type=jnp.float32) + b1[...], 0.0)
            t = jnp.maximum(jnp.dot(t, w2[...],
                                    preferred_element_type=jnp.float32) + b2[...], 0.0)
            outs.append(jnp.dot(t, w3[...],
                                preferred_element_type=jnp.float32) + b3[...])
        out[...] = jnp.concatenate(outs, axis=1)


def _pool(h0, h1, bat, ws0, ws1, bs0, bs1, heads):
    full = lambda shape: pl.BlockSpec(shape, lambda i: tuple(0 for _ in shape))
    head_specs = []
    for w1, b1, w2, b2, w3, b3 in heads:
        head_specs += [full((D, 50)), full((1, 50)), full((50, 25)),
                       full((1, 25)), full((25, 10)), full((1, 10))]
    head_args = [a for h in heads for a in h]
    return pl.pallas_call(
        _pool_body,
        grid=(NB,),
        in_specs=[
            pl.BlockSpec((BLK, DH), lambda i: (i, 0)),
            pl.BlockSpec((BLK, DH), lambda i: (i, 0)),
            pl.BlockSpec((BLK, 1), lambda i: (i, 0)),
            full((D, D)), full((D, D)), full((1, D)), full((1, D)),
        ] + head_specs,
        out_specs=pl.BlockSpec((G, 30), lambda i: (0, 0)),
        out_shape=jax.ShapeDtypeStruct((G, 30), jnp.float32),
        scratch_shapes=[
            pltpu.VMEM((G, D), jnp.float32),
            pltpu.VMEM((G, 1), jnp.float32),
        ],
    )(h0, h1, bat, ws0, ws1, bs0, bs1, *head_args)


def kernel(x, edge_index, batch, Wroot, Wnbr, bconv, gamma, beta,
           Ws, bs, hW1, hb1, hW2, hb2, hW3, hb3):
    src4 = edge_index[0].reshape(NS, NGRP, NCG, CH)
    dst4 = edge_index[1].reshape(NS, NGRP, NCG, CH)
    dst3 = edge_index[1].reshape(NS, NCHUNK, CH)
    h0 = x[:, :DH]
    h1 = x[:, DH:]

    dinv = _deg_kernel(dst3).reshape(N_PAD, 1)

    for i in range(LAYERS):
        m0, m1 = _msg_kernel(h0, h1, src4, dst4)
        wcat = jnp.concatenate([Wroot[i], Wnbr[i]], axis=0)
        z, ssum, ssq = _layer_a(h0, h1, m0, m1, dinv, wcat,
                                bconv[i].reshape(1, D))
        h0, h1 = _layer_b(z, ssum, ssq, gamma[i].reshape(1, D),
                          beta[i].reshape(1, D))

    heads = tuple(
        (hW1[k], hb1[k].reshape(1, 50), hW2[k], hb2[k].reshape(1, 25),
         hW3[k], hb3[k].reshape(1, 10)) for k in range(3))
    return _pool(h0, h1, batch.reshape(N, 1), Ws[0], Ws[1],
                 bs[0].reshape(1, D), bs[1].reshape(1, D), heads)


# speedup vs baseline: 5.2656x; 1.2141x over previous
import functools

import jax
import jax.numpy as jnp
from jax import lax
from jax.experimental import pallas as pl
from jax.experimental.pallas import tpu as pltpu
from jax.experimental.pallas import tpu_sc as plsc

N = 10000
E = 160000
D = 256
DH = D // 2
G = 64
LAYERS = 3

NC = 2
NS = 16
LANES = 16

N_PAD = 10240
ROWS_PT = N_PAD // NS
EPT = E // NS
CH = 80
NCHUNK = EPT // CH
NCG = 25
NGRP = NCHUNK // NCG
ZR = 64

BLK = 400
NB = N // BLK

_sc_mesh = plsc.VectorSubcoreMesh(
    core_axis_name="c", subcore_axis_name="s", num_cores=NC, num_subcores=NS)


@functools.partial(
    pl.kernel,
    out_type=jax.ShapeDtypeStruct((N_PAD,), jnp.float32),
    mesh=_sc_mesh,
    compiler_params=pltpu.CompilerParams(needs_layout_passes=False),
    scratch_types=[
        pltpu.VMEM((NCHUNK, CH), jnp.int32),
        pltpu.VMEM((N_PAD,), jnp.float32),
        pltpu.VMEM((NS, ROWS_PT), jnp.float32),
        pltpu.VMEM_SHARED((NS, NS, ROWS_PT), jnp.float32),
    ],
)
def _deg_kernel(dst_hbm, out_hbm, dst_v, deg_v, red_v, stage_sh):
    c = lax.axis_index("c")
    s = lax.axis_index("s")
    zf = jnp.zeros((LANES,), jnp.float32)
    ones = jnp.ones((LANES,), jnp.float32)

    def zbody(i, carry):
        deg_v[pl.ds(i * LANES, LANES)] = zf
        return carry
    lax.fori_loop(0, N_PAD // LANES, zbody, 0)

    pltpu.sync_copy(dst_hbm.at[s], dst_v)

    def ebody(j, carry):
        def inner(k, carry2):
            idx = dst_v[j, pl.ds(k * LANES, LANES)]
            plsc.addupdate_scatter(deg_v, [idx], ones)
            return carry2
        return lax.fori_loop(0, CH // LANES, inner, carry)
    lax.fori_loop(0, NCHUNK, ebody, 0)

    for k in range(NS):
        pltpu.sync_copy(deg_v.at[pl.ds(k * ROWS_PT, ROWS_PT)], stage_sh.at[k, s])
    plsc.subcore_barrier()

    pltpu.sync_copy(stage_sh.at[s], red_v)

    def rbody(j, carry):
        acc = zf
        for r in range(NS):
            acc = acc + red_v[r, pl.ds(j * LANES, LANES)]
        acc = jnp.maximum(acc, 1.0)
        deg_v[pl.ds(j * LANES, LANES)] = 1.0 / acc
        return carry
    lax.fori_loop(0, ROWS_PT // LANES, rbody, 0)

    @pl.when(c == 0)
    def _():
        pltpu.sync_copy(deg_v.at[pl.ds(0, ROWS_PT)],
                        out_hbm.at[pl.ds(s * ROWS_PT, ROWS_PT)])


@functools.partial(
    pl.kernel,
    out_type=(jax.ShapeDtypeStruct((N_PAD, DH), jnp.float32),
              jax.ShapeDtypeStruct((N_PAD, DH), jnp.float32)),
    mesh=_sc_mesh,
    compiler_params=pltpu.CompilerParams(needs_layout_passes=False),
    scratch_types=[
        pltpu.VMEM((NCG, CH), jnp.int32),
        pltpu.VMEM((NCG, CH), jnp.int32),
        pltpu.VMEM((CH, DH), jnp.float32),
        pltpu.VMEM((CH, DH), jnp.float32),
        pltpu.VMEM((ZR, DH), jnp.float32),
        pltpu.VMEM_SHARED((N_PAD, DH), jnp.float32),
        pltpu.SemaphoreType.DMA,
        pltpu.SemaphoreType.DMA,
    ],
)
def _msg_kernel(h0_hbm, h1_hbm, src_hbm, dst_hbm, out0_hbm, out1_hbm,
                src_v, dst_v, rows_a, rows_b, zero_v, acc_sh, sga, sgb):
    c = lax.axis_index("c")
    s = lax.axis_index("s")
    zf = jnp.zeros((LANES,), jnp.float32)

    def zrow(i, carry):
        def zcol(j, carry2):
            zero_v[i, pl.ds(j * LANES, LANES)] = zf
            return carry2
        return lax.fori_loop(0, DH // LANES, zcol, carry)
    lax.fori_loop(0, ZR, zrow, 0)

    base = s * ROWS_PT
    for k in range(ROWS_PT // ZR):
        pltpu.sync_copy(zero_v, acc_sh.at[pl.ds(base + k * ZR, ZR)])

    plsc.subcore_barrier()

    def make_loop(h_hbm):
        def run():
            for g in range(NGRP):
                pltpu.sync_copy(src_hbm.at[s, g], src_v)
                pltpu.sync_copy(dst_hbm.at[s, g], dst_v)

                pltpu.async_copy(h_hbm.at[src_v.at[0]], rows_a, sga)

                def pair(t, carry):
                    for b in range(2):
                        jj = 2 * t + b
                        buf, sb = (rows_a, sga) if b == 0 else (rows_b, sgb)
                        obuf, ob = (rows_b, sgb) if b == 0 else (rows_a, sga)
                        pltpu.make_async_copy(
                            h_hbm.at[src_v.at[jj]], buf, sb).wait()
                        pltpu.async_copy(h_hbm.at[src_v.at[jj + 1]], obuf, ob)
                        pltpu.sync_copy(buf, acc_sh.at[dst_v.at[jj]], add=True)
                    return carry
                lax.fori_loop(0, (NCG - 1) // 2, pair, 0)

                pltpu.make_async_copy(
                    h_hbm.at[src_v.at[NCG - 1]], rows_a, sga).wait()
                pltpu.sync_copy(rows_a, acc_sh.at[dst_v.at[NCG - 1]], add=True)
        return run

    @pl.when(c == 0)
    def _():
        make_loop(h0_hbm)()

    @pl.when(c == 1)
    def _():
        make_loop(h1_hbm)()

    plsc.subcore_barrier()

    @pl.when(c == 0)
    def _():
        pltpu.sync_copy(acc_sh.at[pl.ds(base, ROWS_PT)],
                        out0_hbm.at[pl.ds(base, ROWS_PT)])

    @pl.when(c == 1)
    def _():
        pltpu.sync_copy(acc_sh.at[pl.ds(base, ROWS_PT)],
                        out1_hbm.at[pl.ds(base, ROWS_PT)])


def _layer_a_body(h0, h1, m0, m1, dinv, wcat, bc, z, ssum, ssq):
    i = pl.program_id(0)
    di = dinv[...]
    hm = jnp.concatenate(
        [h0[...], h1[...], m0[...] * di, m1[...] * di], axis=1)
    zz = jnp.dot(hm, wcat[...], preferred_element_type=jnp.float32) + bc[...]
    z[...] = zz

    @pl.when(i == 0)
    def _():
        ssum[...] = jnp.zeros_like(ssum)
        ssq[...] = jnp.zeros_like(ssq)

    ssum[...] += jnp.sum(zz, axis=0, keepdims=True)
    ssq[...] += jnp.sum(zz * zz, axis=0, keepdims=True)


def _layer_a(h0, h1, m0, m1, dinv, wcat, bc):
    return pl.pallas_call(
        _layer_a_body,
        grid=(NB,),
        in_specs=[
            pl.BlockSpec((BLK, DH), lambda i: (i, 0)),
            pl.BlockSpec((BLK, DH), lambda i: (i, 0)),
            pl.BlockSpec((BLK, DH), lambda i: (i, 0)),
            pl.BlockSpec((BLK, DH), lambda i: (i, 0)),
            pl.BlockSpec((BLK, 1), lambda i: (i, 0)),
            pl.BlockSpec((2 * D, D), lambda i: (0, 0)),
            pl.BlockSpec((1, D), lambda i: (0, 0)),
        ],
        out_specs=[
            pl.BlockSpec((BLK, D), lambda i: (i, 0)),
            pl.BlockSpec((1, D), lambda i: (0, 0)),
            pl.BlockSpec((1, D), lambda i: (0, 0)),
        ],
        out_shape=[
            jax.ShapeDtypeStruct((N, D), jnp.float32),
            jax.ShapeDtypeStruct((1, D), jnp.float32),
            jax.ShapeDtypeStruct((1, D), jnp.float32),
        ],
    )(h0, h1, m0, m1, dinv, wcat, bc)


def _layer_b_body(z, ssum, ssq, gam, bet, h0, h1):
    mu = ssum[...] * (1.0 / N)
    var = ssq[...] * (1.0 / N) - mu * mu
    y = (z[...] - mu) / jnp.sqrt(var + 1e-5) * gam[...] + bet[...]
    y = jnp.maximum(y, 0.0)
    h0[...] = y[:, :DH]
    h1[...] = y[:, DH:]


def _layer_b(z, ssum, ssq, gam, bet):
    return pl.pallas_call(
        _layer_b_body,
        grid=(NB,),
        in_specs=[
            pl.BlockSpec((BLK, D), lambda i: (i, 0)),
            pl.BlockSpec((1, D), lambda i: (0, 0)),
            pl.BlockSpec((1, D), lambda i: (0, 0)),
            pl.BlockSpec((1, D), lambda i: (0, 0)),
            pl.BlockSpec((1, D), lambda i: (0, 0)),
        ],
        out_specs=[
            pl.BlockSpec((BLK, DH), lambda i: (i, 0)),
            pl.BlockSpec((BLK, DH), lambda i: (i, 0)),
        ],
        out_shape=[
            jax.ShapeDtypeStruct((N, DH), jnp.float32),
            jax.ShapeDtypeStruct((N, DH), jnp.float32),
        ],
    )(z, ssum, ssq, gam, bet)


def _pool_body(h0, h1, bat, ws0, ws1, bs0, bs1,
               w10, b10, w20, b20, w30, b30,
               w11, b11, w21, b21, w31, b31,
               w12, b12, w22, b22, w32, b32,
               out, gsum, gcnt):
    i = pl.program_id(0)

    @pl.when(i == 0)
    def _():
        gsum[...] = jnp.zeros_like(gsum)
        gcnt[...] = jnp.zeros_like(gcnt)

    h = jnp.concatenate([h0[...], h1[...]], axis=1)
    onehot = (bat[...] == lax.broadcasted_iota(jnp.int32, (1, G), 1)
              ).astype(jnp.float32)
    gsum[...] += lax.dot_general(onehot, h, (((0,), (0,)), ((), ())),
                                 preferred_element_type=jnp.float32)
    gcnt[...] += lax.dot_general(onehot, jnp.ones((BLK, 1), jnp.float32),
                                 (((0,), (0,)), ((), ())),
                                 preferred_element_type=jnp.float32)

    @pl.when(i == NB - 1)
    def _():
        g = gsum[...] / jnp.maximum(gcnt[...], 1.0)
        g = jnp.maximum(jnp.dot(g, ws0[...],
                                preferred_element_type=jnp.float32) + bs0[...], 0.0)
        g = jnp.maximum(jnp.dot(g, ws1[...],
                                preferred_element_type=jnp.float32) + bs1[...], 0.0)
        outs = []
        for w1, b1, w2, b2, w3, b3 in ((w10, b10, w20, b20, w30, b30),
                                       (w11, b11, w21, b21, w31, b31),
                                       (w12, b12, w22, b22, w32, b32)):
            t = jnp.maximum(jnp.dot(g, w1[...],
                                    preferred_element_type=jnp.float32) + b1[...], 0.0)
            t = jnp.maximum(jnp.dot(t, w2[...],
                                    preferred_element_type=jnp.float32) + b2[...], 0.0)
            outs.append(jnp.dot(t, w3[...],
                                preferred_element_type=jnp.float32) + b3[...])
        out[...] = jnp.concatenate(outs, axis=1)


def _pool(h0, h1, bat, ws0, ws1, bs0, bs1, heads):
    full = lambda shape: pl.BlockSpec(shape, lambda i: tuple(0 for _ in shape))
    head_specs = []
    for w1, b1, w2, b2, w3, b3 in heads:
        head_specs += [full((D, 50)), full((1, 50)), full((50, 25)),
                       full((1, 25)), full((25, 10)), full((1, 10))]
    head_args = [a for h in heads for a in h]
    return pl.pallas_call(
        _pool_body,
        grid=(NB,),
        in_specs=[
            pl.BlockSpec((BLK, DH), lambda i: (i, 0)),
            pl.BlockSpec((BLK, DH), lambda i: (i, 0)),
            pl.BlockSpec((BLK, 1), lambda i: (i, 0)),
            full((D, D)), full((D, D)), full((1, D)), full((1, D)),
        ] + head_specs,
        out_specs=pl.BlockSpec((G, 30), lambda i: (0, 0)),
        out_shape=jax.ShapeDtypeStruct((G, 30), jnp.float32),
        scratch_shapes=[
            pltpu.VMEM((G, D), jnp.float32),
            pltpu.VMEM((G, 1), jnp.float32),
        ],
    )(h0, h1, bat, ws0, ws1, bs0, bs1, *head_args)


def kernel(x, edge_index, batch, Wroot, Wnbr, bconv, gamma, beta,
           Ws, bs, hW1, hb1, hW2, hb2, hW3, hb3):
    src4 = edge_index[0].reshape(NS, NGRP, NCG, CH)
    dst4 = edge_index[1].reshape(NS, NGRP, NCG, CH)
    dst3 = edge_index[1].reshape(NS, NCHUNK, CH)
    h0 = x[:, :DH]
    h1 = x[:, DH:]

    dinv = _deg_kernel(dst3).reshape(N_PAD, 1)

    for i in range(LAYERS):
        m0, m1 = _msg_kernel(h0, h1, src4, dst4)
        wcat = jnp.concatenate([Wroot[i], Wnbr[i]], axis=0)
        z, ssum, ssq = _layer_a(h0, h1, m0, m1, dinv, wcat,
                                bconv[i].reshape(1, D))
        h0, h1 = _layer_b(z, ssum, ssq, gamma[i].reshape(1, D),
                          beta[i].reshape(1, D))

    heads = tuple(
        (hW1[k], hb1[k].reshape(1, 50), hW2[k], hb2[k].reshape(1, 25),
         hW3[k], hb3[k].reshape(1, 10)) for k in range(3))
    return _pool(h0, h1, batch.reshape(N, 1), Ws[0], Ws[1],
                 bs[0].reshape(1, D), bs[1].reshape(1, D), heads)


# speedup vs baseline: 6.7708x; 1.2859x over previous
import functools

import jax
import jax.numpy as jnp
from jax import lax
from jax.experimental import pallas as pl
from jax.experimental.pallas import tpu as pltpu
from jax.experimental.pallas import tpu_sc as plsc

N = 10000
E = 160000
D = 256
DH = D // 2
G = 64
LAYERS = 3

NC = 2
NS = 16
LANES = 16

N_PAD = 10240
ROWS_PT = N_PAD // NS
EPT = E // NS
CH = 80
NCHUNK = EPT // CH
NCG = 25
NGRP = NCHUNK // NCG
ZR = 32

BLK = 400
NB = N // BLK

_sc_mesh = plsc.VectorSubcoreMesh(
    core_axis_name="c", subcore_axis_name="s", num_cores=NC, num_subcores=NS)


@functools.partial(
    pl.kernel,
    out_type=jax.ShapeDtypeStruct((N_PAD,), jnp.float32),
    mesh=_sc_mesh,
    compiler_params=pltpu.CompilerParams(needs_layout_passes=False),
    scratch_types=[
        pltpu.VMEM((NCHUNK, CH), jnp.int32),
        pltpu.VMEM((N_PAD,), jnp.float32),
        pltpu.VMEM((NS, ROWS_PT), jnp.float32),
        pltpu.VMEM_SHARED((NS, NS, ROWS_PT), jnp.float32),
    ],
)
def _deg_kernel(dst_hbm, out_hbm, dst_v, deg_v, red_v, stage_sh):
    c = lax.axis_index("c")
    s = lax.axis_index("s")
    zf = jnp.zeros((LANES,), jnp.float32)
    ones = jnp.ones((LANES,), jnp.float32)

    def zbody(i, carry):
        deg_v[pl.ds(i * LANES, LANES)] = zf
        return carry
    lax.fori_loop(0, N_PAD // LANES, zbody, 0)

    pltpu.sync_copy(dst_hbm.at[s], dst_v)

    def ebody(j, carry):
        def inner(k, carry2):
            idx = dst_v[j, pl.ds(k * LANES, LANES)]
            plsc.addupdate_scatter(deg_v, [idx], ones)
            return carry2
        return lax.fori_loop(0, CH // LANES, inner, carry)
    lax.fori_loop(0, NCHUNK, ebody, 0)

    for k in range(NS):
        pltpu.sync_copy(deg_v.at[pl.ds(k * ROWS_PT, ROWS_PT)], stage_sh.at[k, s])
    plsc.subcore_barrier()

    pltpu.sync_copy(stage_sh.at[s], red_v)

    def rbody(j, carry):
        acc = zf
        for r in range(NS):
            acc = acc + red_v[r, pl.ds(j * LANES, LANES)]
        acc = jnp.maximum(acc, 1.0)
        deg_v[pl.ds(j * LANES, LANES)] = 1.0 / acc
        return carry
    lax.fori_loop(0, ROWS_PT // LANES, rbody, 0)

    @pl.when(c == 0)
    def _():
        pltpu.sync_copy(deg_v.at[pl.ds(0, ROWS_PT)],
                        out_hbm.at[pl.ds(s * ROWS_PT, ROWS_PT)])


@functools.partial(
    pl.kernel,
    out_type=(jax.ShapeDtypeStruct((N_PAD, DH), jnp.float32),
              jax.ShapeDtypeStruct((N_PAD, DH), jnp.float32)),
    mesh=_sc_mesh,
    compiler_params=pltpu.CompilerParams(needs_layout_passes=False),
    scratch_types=[
        pltpu.VMEM((NCG, CH), jnp.int32),
        pltpu.VMEM((NCG, CH), jnp.int32),
        pltpu.VMEM((CH, DH), jnp.float32),
        pltpu.VMEM((CH, DH), jnp.float32),
        pltpu.VMEM((CH, DH), jnp.float32),
        pltpu.VMEM((ZR, DH), jnp.float32),
        pltpu.VMEM_SHARED((N_PAD, DH), jnp.float32),
        pltpu.SemaphoreType.DMA,
        pltpu.SemaphoreType.DMA,
        pltpu.SemaphoreType.DMA,
    ],
)
def _msg_kernel(h0_hbm, h1_hbm, src_hbm, dst_hbm, out0_hbm, out1_hbm,
                src_v, dst_v, rows_a, rows_b, rows_c, zero_v, acc_sh,
                sga, sgb, sgc):
    c = lax.axis_index("c")
    s = lax.axis_index("s")
    zf = jnp.zeros((LANES,), jnp.float32)

    def zrow(i, carry):
        def zcol(j, carry2):
            zero_v[i, pl.ds(j * LANES, LANES)] = zf
            return carry2
        return lax.fori_loop(0, DH // LANES, zcol, carry)
    lax.fori_loop(0, ZR, zrow, 0)

    base = s * ROWS_PT
    for k in range(ROWS_PT // ZR):
        pltpu.sync_copy(zero_v, acc_sh.at[pl.ds(base + k * ZR, ZR)])

    plsc.subcore_barrier()

    def make_loop(h_hbm):
        bufs = (rows_a, rows_b, rows_c)
        sems = (sga, sgb, sgc)

        def run():
            for g in range(NGRP):
                pltpu.sync_copy(src_hbm.at[s, g], src_v)
                pltpu.sync_copy(dst_hbm.at[s, g], dst_v)

                pltpu.async_copy(h_hbm.at[src_v.at[0]], rows_a, sga)
                pltpu.async_copy(h_hbm.at[src_v.at[1]], rows_b, sgb)

                def triple(t, carry):
                    for b in range(3):
                        jj = 3 * t + b
                        buf, sb = bufs[b], sems[b]
                        nb = (b + 2) % 3
                        pltpu.make_async_copy(
                            h_hbm.at[src_v.at[jj]], buf, sb).wait()

                        @pl.when(jj + 2 < NCG)
                        def _():
                            pltpu.async_copy(
                                h_hbm.at[src_v.at[jj + 2]], bufs[nb], sems[nb])
                        pltpu.sync_copy(buf, acc_sh.at[dst_v.at[jj]], add=True)
                    return carry
                lax.fori_loop(0, (NCG - 1) // 3, triple, 0)

                jl = NCG - 1
                pltpu.make_async_copy(
                    h_hbm.at[src_v.at[jl]], bufs[jl % 3], sems[jl % 3]).wait()
                pltpu.sync_copy(bufs[jl % 3],
                                acc_sh.at[dst_v.at[jl]], add=True)
        return run

    @pl.when(c == 0)
    def _():
        make_loop(h0_hbm)()

    @pl.when(c == 1)
    def _():
        make_loop(h1_hbm)()

    plsc.subcore_barrier()

    @pl.when(c == 0)
    def _():
        pltpu.sync_copy(acc_sh.at[pl.ds(base, ROWS_PT)],
                        out0_hbm.at[pl.ds(base, ROWS_PT)])

    @pl.when(c == 1)
    def _():
        pltpu.sync_copy(acc_sh.at[pl.ds(base, ROWS_PT)],
                        out1_hbm.at[pl.ds(base, ROWS_PT)])


def _layer_a_body(h0, h1, m0, m1, dinv, wcat, bc, z, ssum, ssq):
    i = pl.program_id(0)
    di = dinv[...]
    hm = jnp.concatenate(
        [h0[...], h1[...], m0[...] * di, m1[...] * di], axis=1)
    zz = jnp.dot(hm, wcat[...], preferred_element_type=jnp.float32) + bc[...]
    z[...] = zz

    @pl.when(i == 0)
    def _():
        ssum[...] = jnp.zeros_like(ssum)
        ssq[...] = jnp.zeros_like(ssq)

    ssum[...] += jnp.sum(zz, axis=0, keepdims=True)
    ssq[...] += jnp.sum(zz * zz, axis=0, keepdims=True)


def _layer_a(h0, h1, m0, m1, dinv, wcat, bc):
    return pl.pallas_call(
        _layer_a_body,
        grid=(NB,),
        in_specs=[
            pl.BlockSpec((BLK, DH), lambda i: (i, 0)),
            pl.BlockSpec((BLK, DH), lambda i: (i, 0)),
            pl.BlockSpec((BLK, DH), lambda i: (i, 0)),
            pl.BlockSpec((BLK, DH), lambda i: (i, 0)),
            pl.BlockSpec((BLK, 1), lambda i: (i, 0)),
            pl.BlockSpec((2 * D, D), lambda i: (0, 0)),
            pl.BlockSpec((1, D), lambda i: (0, 0)),
        ],
        out_specs=[
            pl.BlockSpec((BLK, D), lambda i: (i, 0)),
            pl.BlockSpec((1, D), lambda i: (0, 0)),
            pl.BlockSpec((1, D), lambda i: (0, 0)),
        ],
        out_shape=[
            jax.ShapeDtypeStruct((N, D), jnp.float32),
            jax.ShapeDtypeStruct((1, D), jnp.float32),
            jax.ShapeDtypeStruct((1, D), jnp.float32),
        ],
    )(h0, h1, m0, m1, dinv, wcat, bc)


def _layer_b_body(z, ssum, ssq, gam, bet, h0, h1):
    mu = ssum[...] * (1.0 / N)
    var = ssq[...] * (1.0 / N) - mu * mu
    y = (z[...] - mu) / jnp.sqrt(var + 1e-5) * gam[...] + bet[...]
    y = jnp.maximum(y, 0.0)
    h0[...] = y[:, :DH]
    h1[...] = y[:, DH:]


def _layer_b(z, ssum, ssq, gam, bet):
    return pl.pallas_call(
        _layer_b_body,
        grid=(NB,),
        in_specs=[
            pl.BlockSpec((BLK, D), lambda i: (i, 0)),
            pl.BlockSpec((1, D), lambda i: (0, 0)),
            pl.BlockSpec((1, D), lambda i: (0, 0)),
            pl.BlockSpec((1, D), lambda i: (0, 0)),
            pl.BlockSpec((1, D), lambda i: (0, 0)),
        ],
        out_specs=[
            pl.BlockSpec((BLK, DH), lambda i: (i, 0)),
            pl.BlockSpec((BLK, DH), lambda i: (i, 0)),
        ],
        out_shape=[
            jax.ShapeDtypeStruct((N, DH), jnp.float32),
            jax.ShapeDtypeStruct((N, DH), jnp.float32),
        ],
    )(z, ssum, ssq, gam, bet)


def _pool_body(h0, h1, bat, ws0, ws1, bs0, bs1,
               w10, b10, w20, b20, w30, b30,
               w11, b11, w21, b21, w31, b31,
               w12, b12, w22, b22, w32, b32,
               out, gsum, gcnt):
    i = pl.program_id(0)

    @pl.when(i == 0)
    def _():
        gsum[...] = jnp.zeros_like(gsum)
        gcnt[...] = jnp.zeros_like(gcnt)

    h = jnp.concatenate([h0[...], h1[...]], axis=1)
    onehot = (bat[...] == lax.broadcasted_iota(jnp.int32, (1, G), 1)
              ).astype(jnp.float32)
    gsum[...] += lax.dot_general(onehot, h, (((0,), (0,)), ((), ())),
                                 preferred_element_type=jnp.float32)
    gcnt[...] += lax.dot_general(onehot, jnp.ones((BLK, 1), jnp.float32),
                                 (((0,), (0,)), ((), ())),
                                 preferred_element_type=jnp.float32)

    @pl.when(i == NB - 1)
    def _():
        g = gsum[...] / jnp.maximum(gcnt[...], 1.0)
        g = jnp.maximum(jnp.dot(g, ws0[...],
                                preferred_element_type=jnp.float32) + bs0[...], 0.0)
        g = jnp.maximum(jnp.dot(g, ws1[...],
                                preferred_element_type=jnp.float32) + bs1[...], 0.0)
        outs = []
        for w1, b1, w2, b2, w3, b3 in ((w10, b10, w20, b20, w30, b30),
                                       (w11, b11, w21, b21, w31, b31),
                                       (w12, b12, w22, b22, w32, b32)):
            t = jnp.maximum(jnp.dot(g, w1[...],
                                    preferred_element_type=jnp.float32) + b1[...], 0.0)
            t = jnp.maximum(jnp.dot(t, w2[...],
                                    preferred_element_type=jnp.float32) + b2[...], 0.0)
            outs.append(jnp.dot(t, w3[...],
                                preferred_element_type=jnp.float32) + b3[...])
        out[...] = jnp.concatenate(outs, axis=1)


def _pool(h0, h1, bat, ws0, ws1, bs0, bs1, heads):
    full = lambda shape: pl.BlockSpec(shape, lambda i: tuple(0 for _ in shape))
    head_specs = []
    for w1, b1, w2, b2, w3, b3 in heads:
        head_specs += [full((D, 50)), full((1, 50)), full((50, 25)),
                       full((1, 25)), full((25, 10)), full((1, 10))]
    head_args = [a for h in heads for a in h]
    return pl.pallas_call(
        _pool_body,
        grid=(NB,),
        in_specs=[
            pl.BlockSpec((BLK, DH), lambda i: (i, 0)),
            pl.BlockSpec((BLK, DH), lambda i: (i, 0)),
            pl.BlockSpec((BLK, 1), lambda i: (i, 0)),
            full((D, D)), full((D, D)), full((1, D)), full((1, D)),
        ] + head_specs,
        out_specs=pl.BlockSpec((G, 30), lambda i: (0, 0)),
        out_shape=jax.ShapeDtypeStruct((G, 30), jnp.float32),
        scratch_shapes=[
            pltpu.VMEM((G, D), jnp.float32),
            pltpu.VMEM((G, 1), jnp.float32),
        ],
    )(h0, h1, bat, ws0, ws1, bs0, bs1, *head_args)


def kernel(x, edge_index, batch, Wroot, Wnbr, bconv, gamma, beta,
           Ws, bs, hW1, hb1, hW2, hb2, hW3, hb3):
    src4 = edge_index[0].reshape(NS, NGRP, NCG, CH)
    dst4 = edge_index[1].reshape(NS, NGRP, NCG, CH)
    dst3 = edge_index[1].reshape(NS, NCHUNK, CH)
    h0 = x[:, :DH]
    h1 = x[:, DH:]

    dinv = _deg_kernel(dst3).reshape(N_PAD, 1)

    for i in range(LAYERS):
        m0, m1 = _msg_kernel(h0, h1, src4, dst4)
        wcat = jnp.concatenate([Wroot[i], Wnbr[i]], axis=0)
        z, ssum, ssq = _layer_a(h0, h1, m0, m1, dinv, wcat,
                                bconv[i].reshape(1, D))
        h0, h1 = _layer_b(z, ssum, ssq, gamma[i].reshape(1, D),
                          beta[i].reshape(1, D))

    heads = tuple(
        (hW1[k], hb1[k].reshape(1, 50), hW2[k], hb2[k].reshape(1, 25),
         hW3[k], hb3[k].reshape(1, 10)) for k in range(3))
    return _pool(h0, h1, batch.reshape(N, 1), Ws[0], Ws[1],
                 bs[0].reshape(1, D), bs[1].reshape(1, D), heads)


# speedup vs baseline: 6.9348x; 1.0242x over previous
import functools

import jax
import jax.numpy as jnp
from jax import lax
from jax.experimental import pallas as pl
from jax.experimental.pallas import tpu as pltpu
from jax.experimental.pallas import tpu_sc as plsc

N = 10000
E = 160000
D = 256
DH = D // 2
G = 64
LAYERS = 3

NC = 2
NS = 16
LANES = 16

N_PAD = 10240
ROWS_PT = N_PAD // NS
EPT = E // NS
CH = 80
NCHUNK = EPT // CH
NCG = 25
NGRP = NCHUNK // NCG
ZR = 16
ROWS_FULL = 640
ROWS_LAST = N - 15 * ROWS_FULL

BLK = 400
NB = N // BLK

_sc_mesh = plsc.VectorSubcoreMesh(
    core_axis_name="c", subcore_axis_name="s", num_cores=NC, num_subcores=NS)


@functools.partial(
    pl.kernel,
    out_type=jax.ShapeDtypeStruct((N_PAD,), jnp.float32),
    mesh=_sc_mesh,
    compiler_params=pltpu.CompilerParams(needs_layout_passes=False),
    scratch_types=[
        pltpu.VMEM((NCHUNK, CH), jnp.int32),
        pltpu.VMEM((N_PAD,), jnp.float32),
        pltpu.VMEM((NS, ROWS_PT), jnp.float32),
        pltpu.VMEM_SHARED((NS, NS, ROWS_PT), jnp.float32),
    ],
)
def _deg_kernel(dst_hbm, out_hbm, dst_v, deg_v, red_v, stage_sh):
    c = lax.axis_index("c")
    s = lax.axis_index("s")
    zf = jnp.zeros((LANES,), jnp.float32)
    ones = jnp.ones((LANES,), jnp.float32)

    def zbody(i, carry):
        deg_v[pl.ds(i * LANES, LANES)] = zf
        return carry
    lax.fori_loop(0, N_PAD // LANES, zbody, 0)

    pltpu.sync_copy(dst_hbm.at[s], dst_v)

    def ebody(j, carry):
        def inner(k, carry2):
            idx = dst_v[j, pl.ds(k * LANES, LANES)]
            plsc.addupdate_scatter(deg_v, [idx], ones)
            return carry2
        return lax.fori_loop(0, CH // LANES, inner, carry)
    lax.fori_loop(0, NCHUNK, ebody, 0)

    for k in range(NS):
        pltpu.sync_copy(deg_v.at[pl.ds(k * ROWS_PT, ROWS_PT)], stage_sh.at[k, s])
    plsc.subcore_barrier()

    pltpu.sync_copy(stage_sh.at[s], red_v)

    def rbody(j, carry):
        acc = zf
        for r in range(NS):
            acc = acc + red_v[r, pl.ds(j * LANES, LANES)]
        acc = jnp.maximum(acc, 1.0)
        deg_v[pl.ds(j * LANES, LANES)] = 1.0 / acc
        return carry
    lax.fori_loop(0, ROWS_PT // LANES, rbody, 0)

    @pl.when(c == 0)
    def _():
        pltpu.sync_copy(deg_v.at[pl.ds(0, ROWS_PT)],
                        out_hbm.at[pl.ds(s * ROWS_PT, ROWS_PT)])


@functools.partial(
    pl.kernel,
    out_type=(jax.ShapeDtypeStruct((N_PAD, DH), jnp.float32),
              jax.ShapeDtypeStruct((N_PAD, DH), jnp.float32)),
    mesh=_sc_mesh,
    compiler_params=pltpu.CompilerParams(needs_layout_passes=False),
    scratch_types=[
        pltpu.VMEM((NCG, CH), jnp.int32),
        pltpu.VMEM((NCG, CH), jnp.int32),
        pltpu.VMEM((CH, DH), jnp.float32),
        pltpu.VMEM((CH, DH), jnp.float32),
        pltpu.VMEM((CH, DH), jnp.float32),
        pltpu.VMEM((CH, DH), jnp.float32),
        pltpu.VMEM_SHARED((N, DH), jnp.float32),
        pltpu.SemaphoreType.DMA,
        pltpu.SemaphoreType.DMA,
        pltpu.SemaphoreType.DMA,
        pltpu.SemaphoreType.DMA,
    ],
)
def _msg_kernel(h0_hbm, h1_hbm, src_hbm, dst_hbm, out0_hbm, out1_hbm,
                src_v, dst_v, rows_a, rows_b, rows_c, rows_d, acc_sh,
                sga, sgb, sgc, sgd):
    c = lax.axis_index("c")
    s = lax.axis_index("s")
    zf = jnp.zeros((LANES,), jnp.float32)

    def zrow(i, carry):
        def zcol(j, carry2):
            rows_a[i, pl.ds(j * LANES, LANES)] = zf
            return carry2
        return lax.fori_loop(0, DH // LANES, zcol, carry)
    lax.fori_loop(0, CH, zrow, 0)

    base = s * ROWS_FULL

    @pl.when(s < 15)
    def _():
        for k in range(ROWS_FULL // CH):
            pltpu.sync_copy(rows_a, acc_sh.at[pl.ds(base + k * CH, CH)])

    @pl.when(s == 15)
    def _():
        for k in range(ROWS_LAST // CH):
            pltpu.sync_copy(rows_a,
                            acc_sh.at[pl.ds(15 * ROWS_FULL + k * CH, CH)])

    plsc.subcore_barrier()

    def make_loop(h_hbm):
        bufs = (rows_a, rows_b, rows_c, rows_d)
        sems = (sga, sgb, sgc, sgd)

        def run():
            for g in range(NGRP):
                pltpu.sync_copy(src_hbm.at[s, g], src_v)
                pltpu.sync_copy(dst_hbm.at[s, g], dst_v)

                pltpu.async_copy(h_hbm.at[src_v.at[0]], rows_a, sga)
                pltpu.async_copy(h_hbm.at[src_v.at[1]], rows_b, sgb)
                pltpu.async_copy(h_hbm.at[src_v.at[2]], rows_c, sgc)

                def quad(t, carry):
                    for b in range(4):
                        jj = 4 * t + b
                        buf, sb = bufs[b], sems[b]
                        nb = (b + 3) % 4
                        pltpu.make_async_copy(
                            h_hbm.at[src_v.at[jj]], buf, sb).wait()

                        @pl.when(jj + 3 < NCG)
                        def _():
                            pltpu.async_copy(
                                h_hbm.at[src_v.at[jj + 3]], bufs[nb], sems[nb])
                        pltpu.sync_copy(buf, acc_sh.at[dst_v.at[jj]], add=True)
                    return carry
                lax.fori_loop(0, (NCG - 1) // 4, quad, 0)

                jl = NCG - 1
                pltpu.make_async_copy(
                    h_hbm.at[src_v.at[jl]], bufs[jl % 4], sems[jl % 4]).wait()
                pltpu.sync_copy(bufs[jl % 4],
                                acc_sh.at[dst_v.at[jl]], add=True)
        return run

    @pl.when(c == 0)
    def _():
        make_loop(h0_hbm)()

    @pl.when(c == 1)
    def _():
        make_loop(h1_hbm)()

    plsc.subcore_barrier()

    def copy_out(out_hbm):
        def run():
            @pl.when(s < 15)
            def _():
                pltpu.sync_copy(acc_sh.at[pl.ds(base, ROWS_FULL)],
                                out_hbm.at[pl.ds(base, ROWS_FULL)])

            @pl.when(s == 15)
            def _():
                pltpu.sync_copy(acc_sh.at[pl.ds(15 * ROWS_FULL, ROWS_LAST)],
                                out_hbm.at[pl.ds(15 * ROWS_FULL, ROWS_LAST)])
        return run

    @pl.when(c == 0)
    def _():
        copy_out(out0_hbm)()

    @pl.when(c == 1)
    def _():
        copy_out(out1_hbm)()


def _layer_a_body(h0, h1, m0, m1, dinv, wcat, bc, z, ssum, ssq):
    i = pl.program_id(0)
    di = dinv[...]
    hm = jnp.concatenate(
        [h0[...], h1[...], m0[...] * di, m1[...] * di], axis=1)
    zz = jnp.dot(hm, wcat[...], preferred_element_type=jnp.float32) + bc[...]
    z[...] = zz

    @pl.when(i == 0)
    def _():
        ssum[...] = jnp.zeros_like(ssum)
        ssq[...] = jnp.zeros_like(ssq)

    ssum[...] += jnp.sum(zz, axis=0, keepdims=True)
    ssq[...] += jnp.sum(zz * zz, axis=0, keepdims=True)


def _layer_a(h0, h1, m0, m1, dinv, wcat, bc):
    return pl.pallas_call(
        _layer_a_body,
        grid=(NB,),
        in_specs=[
            pl.BlockSpec((BLK, DH), lambda i: (i, 0)),
            pl.BlockSpec((BLK, DH), lambda i: (i, 0)),
            pl.BlockSpec((BLK, DH), lambda i: (i, 0)),
            pl.BlockSpec((BLK, DH), lambda i: (i, 0)),
            pl.BlockSpec((BLK, 1), lambda i: (i, 0)),
            pl.BlockSpec((2 * D, D), lambda i: (0, 0)),
            pl.BlockSpec((1, D), lambda i: (0, 0)),
        ],
        out_specs=[
            pl.BlockSpec((BLK, D), lambda i: (i, 0)),
            pl.BlockSpec((1, D), lambda i: (0, 0)),
            pl.BlockSpec((1, D), lambda i: (0, 0)),
        ],
        out_shape=[
            jax.ShapeDtypeStruct((N, D), jnp.float32),
            jax.ShapeDtypeStruct((1, D), jnp.float32),
            jax.ShapeDtypeStruct((1, D), jnp.float32),
        ],
    )(h0, h1, m0, m1, dinv, wcat, bc)


def _layer_b_body(z, ssum, ssq, gam, bet, h0, h1):
    mu = ssum[...] * (1.0 / N)
    var = ssq[...] * (1.0 / N) - mu * mu
    y = (z[...] - mu) / jnp.sqrt(var + 1e-5) * gam[...] + bet[...]
    y = jnp.maximum(y, 0.0)
    h0[...] = y[:, :DH]
    h1[...] = y[:, DH:]


def _layer_b(z, ssum, ssq, gam, bet):
    return pl.pallas_call(
        _layer_b_body,
        grid=(NB,),
        in_specs=[
            pl.BlockSpec((BLK, D), lambda i: (i, 0)),
            pl.BlockSpec((1, D), lambda i: (0, 0)),
            pl.BlockSpec((1, D), lambda i: (0, 0)),
            pl.BlockSpec((1, D), lambda i: (0, 0)),
            pl.BlockSpec((1, D), lambda i: (0, 0)),
        ],
        out_specs=[
            pl.BlockSpec((BLK, DH), lambda i: (i, 0)),
            pl.BlockSpec((BLK, DH), lambda i: (i, 0)),
        ],
        out_shape=[
            jax.ShapeDtypeStruct((N, DH), jnp.float32),
            jax.ShapeDtypeStruct((N, DH), jnp.float32),
        ],
    )(z, ssum, ssq, gam, bet)


def _pool_body(h0, h1, bat, ws0, ws1, bs0, bs1,
               w10, b10, w20, b20, w30, b30,
               w11, b11, w21, b21, w31, b31,
               w12, b12, w22, b22, w32, b32,
               out, gsum, gcnt):
    i = pl.program_id(0)

    @pl.when(i == 0)
    def _():
        gsum[...] = jnp.zeros_like(gsum)
        gcnt[...] = jnp.zeros_like(gcnt)

    h = jnp.concatenate([h0[...], h1[...]], axis=1)
    onehot = (bat[...] == lax.broadcasted_iota(jnp.int32, (1, G), 1)
              ).astype(jnp.float32)
    gsum[...] += lax.dot_general(onehot, h, (((0,), (0,)), ((), ())),
                                 preferred_element_type=jnp.float32)
    gcnt[...] += lax.dot_general(onehot, jnp.ones((BLK, 1), jnp.float32),
                                 (((0,), (0,)), ((), ())),
                                 preferred_element_type=jnp.float32)

    @pl.when(i == NB - 1)
    def _():
        g = gsum[...] / jnp.maximum(gcnt[...], 1.0)
        g = jnp.maximum(jnp.dot(g, ws0[...],
                                preferred_element_type=jnp.float32) + bs0[...], 0.0)
        g = jnp.maximum(jnp.dot(g, ws1[...],
                                preferred_element_type=jnp.float32) + bs1[...], 0.0)
        outs = []
        for w1, b1, w2, b2, w3, b3 in ((w10, b10, w20, b20, w30, b30),
                                       (w11, b11, w21, b21, w31, b31),
                                       (w12, b12, w22, b22, w32, b32)):
            t = jnp.maximum(jnp.dot(g, w1[...],
                                    preferred_element_type=jnp.float32) + b1[...], 0.0)
            t = jnp.maximum(jnp.dot(t, w2[...],
                                    preferred_element_type=jnp.float32) + b2[...], 0.0)
            outs.append(jnp.dot(t, w3[...],
                                preferred_element_type=jnp.float32) + b3[...])
        out[...] = jnp.concatenate(outs, axis=1)


def _pool(h0, h1, bat, ws0, ws1, bs0, bs1, heads):
    full = lambda shape: pl.BlockSpec(shape, lambda i: tuple(0 for _ in shape))
    head_specs = []
    for w1, b1, w2, b2, w3, b3 in heads:
        head_specs += [full((D, 50)), full((1, 50)), full((50, 25)),
                       full((1, 25)), full((25, 10)), full((1, 10))]
    head_args = [a for h in heads for a in h]
    return pl.pallas_call(
        _pool_body,
        grid=(NB,),
        in_specs=[
            pl.BlockSpec((BLK, DH), lambda i: (i, 0)),
            pl.BlockSpec((BLK, DH), lambda i: (i, 0)),
            pl.BlockSpec((BLK, 1), lambda i: (i, 0)),
            full((D, D)), full((D, D)), full((1, D)), full((1, D)),
        ] + head_specs,
        out_specs=pl.BlockSpec((G, 30), lambda i: (0, 0)),
        out_shape=jax.ShapeDtypeStruct((G, 30), jnp.float32),
        scratch_shapes=[
            pltpu.VMEM((G, D), jnp.float32),
            pltpu.VMEM((G, 1), jnp.float32),
        ],
    )(h0, h1, bat, ws0, ws1, bs0, bs1, *head_args)


def kernel(x, edge_index, batch, Wroot, Wnbr, bconv, gamma, beta,
           Ws, bs, hW1, hb1, hW2, hb2, hW3, hb3):
    src4 = edge_index[0].reshape(NS, NGRP, NCG, CH)
    dst4 = edge_index[1].reshape(NS, NGRP, NCG, CH)
    dst3 = edge_index[1].reshape(NS, NCHUNK, CH)
    h0 = x[:, :DH]
    h1 = x[:, DH:]

    dinv = _deg_kernel(dst3).reshape(N_PAD, 1)

    for i in range(LAYERS):
        m0, m1 = _msg_kernel(h0, h1, src4, dst4)
        wcat = jnp.concatenate([Wroot[i], Wnbr[i]], axis=0)
        z, ssum, ssq = _layer_a(h0, h1, m0, m1, dinv, wcat,
                                bconv[i].reshape(1, D))
        h0, h1 = _layer_b(z, ssum, ssq, gamma[i].reshape(1, D),
                          beta[i].reshape(1, D))

    heads = tuple(
        (hW1[k], hb1[k].reshape(1, 50), hW2[k], hb2[k].reshape(1, 25),
         hW3[k], hb3[k].reshape(1, 10)) for k in range(3))
    return _pool(h0, h1, batch.reshape(N, 1), Ws[0], Ws[1],
                 bs[0].reshape(1, D), bs[1].reshape(1, D), heads)
